# baseline (speedup 1.0000x reference)
import functools

import jax
import jax.numpy as jnp
from jax import lax
from jax.experimental import pallas as pl
from jax.experimental.pallas import tpu as pltpu
from jax.experimental.pallas import tpu_sc as plsc

NN = 10000
DD = 128
HH = 128
CC = 40
CP = 64

NC = 2
NS = 16
NWK = NC * NS
CHK = 64
NP = 10240
EE = 320000
NCH = 160
EPW = NCH * CHK
EP = EPW * NWK
HIST_R = NP // 16


def _sc_mesh():
    return plsc.VectorSubcoreMesh(core_axis_name="c", subcore_axis_name="s",
                                  num_cores=NC, num_subcores=NS)


def _deg_body(dst_hbm, zero_hbm, out_hbm, dst_v, hist_v, sem):
    cid = lax.axis_index("c")
    sid = lax.axis_index("s")
    wid = cid * NS + sid
    pltpu.sync_copy(dst_hbm.at[pl.ds(wid * EPW, EPW)], dst_v)
    pltpu.sync_copy(zero_hbm, hist_v)
    ones = jnp.ones((16,), jnp.float32)

    def body(i, _):
        idx = dst_v[pl.ds(i * 16, 16)]
        plsc.addupdate_scatter(hist_v, [idx], ones)
        return 0

    lax.fori_loop(0, EPW // 16, body, 0)
    pltpu.sync_copy(hist_v, out_hbm.at[wid])


def _degree_hist(dstp, zeros_hist):
    k = pl.kernel(
        _deg_body,
        out_type=jax.ShapeDtypeStruct((NWK, NP), jnp.float32),
        mesh=_sc_mesh(),
        compiler_params=pltpu.CompilerParams(needs_layout_passes=False),
        scratch_types=[
            pltpu.VMEM((EPW,), jnp.int32),
            pltpu.VMEM((NP,), jnp.float32),
            pltpu.SemaphoreType.DMA,
        ],
    )
    return k(dstp, zeros_hist)


NBUF = 4
NDB = 8


def _agg_body(width, table_hbm, src_hbm, dst2_hbm, zero_hbm, out_hbm,
              acc_sh, src_v, dst_v, rows_v, *sems):
    cid = lax.axis_index("c")
    sid = lax.axis_index("s")
    wid = cid * NS + sid
    rows_per_tile = NP // NS
    gsem = sems[:NBUF]
    ssem = sems[NBUF:2 * NBUF]
    dsem = sems[2 * NBUF:]

    pltpu.sync_copy(src_hbm.at[pl.ds(wid * EPW, EPW)], src_v)

    def start_didx(ch, b):
        pltpu.async_copy(dst2_hbm.at[pl.ds(wid * NCH + ch, 1)],
                         dst_v.at[pl.ds(b, 1)], dsem[b])

    def wait_didx(ch, b):
        pltpu.make_async_copy(dst2_hbm.at[pl.ds(wid * NCH + ch, 1)],
                              dst_v.at[pl.ds(b, 1)], dsem[b]).wait()

    def start_gather(ch, b):
        pltpu.async_copy(table_hbm.at[src_v.at[pl.ds(ch * CHK, CHK)]],
                         rows_v.at[b], gsem[b])

    def wait_gather(ch, b):
        pltpu.make_async_copy(table_hbm.at[src_v.at[pl.ds(ch * CHK, CHK)]],
                              rows_v.at[b], gsem[b]).wait()

    def start_scatter(ch, db, b):
        pltpu.async_copy(rows_v.at[b], acc_sh.at[dst_v.at[db]], ssem[b],
                         add=True)

    def wait_scatter(ch, db, b):
        pltpu.make_async_copy(rows_v.at[b], acc_sh.at[dst_v.at[db]],
                              ssem[b]).wait()

    for ch in range(NBUF):
        start_didx(ch, ch)
    start_gather(0, 0)
    start_gather(1, 1)
    pltpu.sync_copy(zero_hbm.at[pl.ds(sid * rows_per_tile, rows_per_tile)],
                    acc_sh.at[pl.ds(sid * rows_per_tile, rows_per_tile)])
    plsc.subcore_barrier()

    def outer(k, _):
        for t in range(NDB):
            j = NDB * k + t
            b = t % NBUF
            b2 = (t + 2) % NBUF

            @pl.when(j + 2 < NCH)
            def _():
                @pl.when(j >= 2)
                def _():
                    wait_scatter(j - 2, (t - 2) % NDB, b2)
                start_gather(j + 2, b2)

            @pl.when(j + NBUF < NCH)
            def _():
                start_didx(j + NBUF, (t + NBUF) % NDB)

            wait_gather(j, b)
            wait_didx(j, t)
            start_scatter(j, t, b)
        return 0

    lax.fori_loop(0, NCH // NDB, outer, 0)
    for ch in range(NCH - NBUF, NCH):
        wait_scatter(ch, ch % NDB, ch % NBUF)

    plsc.subcore_barrier()
    pltpu.sync_copy(acc_sh.at[pl.ds(sid * rows_per_tile, rows_per_tile)],
                    out_hbm.at[cid, pl.ds(sid * rows_per_tile, rows_per_tile)])


def _aggregate(table, srcp, dstp2, zeros_acc, width):
    k = pl.kernel(
        functools.partial(_agg_body, width),
        out_type=jax.ShapeDtypeStruct((NC, NP, width), jnp.float32),
        mesh=_sc_mesh(),
        compiler_params=pltpu.CompilerParams(
            needs_layout_passes=False,
            use_tc_tiling_on_sc=(width % 128 == 0)),
        scratch_types=[
            pltpu.VMEM_SHARED((NP, width), jnp.float32),
            pltpu.VMEM((EPW,), jnp.int32),
            pltpu.VMEM((NDB, CHK), jnp.int32),
            pltpu.VMEM((NBUF, CHK, width), jnp.float32),
        ] + [pltpu.SemaphoreType.DMA] * (2 * NBUF + NDB),
    )
    return k(table, srcp, dstp2, zeros_acc)


def _tc1_body(hist_ref, x_ref, w_ref, dis_ref, ts_ref):
    deg = jnp.sum(hist_ref[...], axis=0, keepdims=True) + 1.0
    dis_c = jnp.transpose(lax.rsqrt(deg))
    dis_ref[...] = dis_c
    xw = jnp.dot(x_ref[...], w_ref[...], preferred_element_type=jnp.float32)
    ts_ref[...] = xw * dis_c


def _tc1(hist, x, W1):
    blk = 1024
    return pl.pallas_call(
        _tc1_body,
        grid=(NP // blk,),
        in_specs=[
            pl.BlockSpec((NWK, blk), lambda i: (0, i)),
            pl.BlockSpec((blk, DD), lambda i: (i, 0)),
            pl.BlockSpec((DD, HH), lambda i: (0, 0)),
        ],
        out_specs=[
            pl.BlockSpec((blk, 1), lambda i: (i, 0)),
            pl.BlockSpec((blk, HH), lambda i: (i, 0)),
        ],
        out_shape=[
            jax.ShapeDtypeStruct((NP, 1), jnp.float32),
            jax.ShapeDtypeStruct((NP, HH), jnp.float32),
        ],
    )(hist, x, W1)


def _mid_body(relu, acc0_ref, acc1_ref, ts_ref, dis_ref, b_ref, w_ref,
              out_ref):
    dis = dis_ref[...]
    h = (acc0_ref[0] + acc1_ref[0] + ts_ref[...]) * dis + b_ref[...]
    if relu:
        h = jnp.maximum(h, 0.0)
    out_ref[...] = jnp.dot(h, w_ref[...],
                           preferred_element_type=jnp.float32) * dis


def _tc_mid(acc, ts, dis_col, b, Wn, relu):
    blk = 1024
    win, wout = Wn.shape
    return pl.pallas_call(
        functools.partial(_mid_body, relu),
        grid=(NP // blk,),
        in_specs=[
            pl.BlockSpec((1, blk, win), lambda i: (0, i, 0)),
            pl.BlockSpec((1, blk, win), lambda i: (1, i, 0)),
            pl.BlockSpec((blk, win), lambda i: (i, 0)),
            pl.BlockSpec((blk, 1), lambda i: (i, 0)),
            pl.BlockSpec((1, win), lambda i: (0, 0)),
            pl.BlockSpec((win, wout), lambda i: (0, 0)),
        ],
        out_specs=pl.BlockSpec((blk, wout), lambda i: (i, 0)),
        out_shape=jax.ShapeDtypeStruct((NP, wout), jnp.float32),
    )(acc, acc, ts, dis_col, b, Wn)


def _out_body(acc0_ref, acc1_ref, ts_ref, dis_ref, b_ref, out_ref):
    hp = (acc0_ref[0] + acc1_ref[0] + ts_ref[...]) * dis_ref[...] + b_ref[...]
    h = hp[:, :CC]
    m = jnp.max(h, axis=1, keepdims=True)
    e = jnp.exp(h - m)
    lse = jnp.log(jnp.sum(e, axis=1, keepdims=True))
    out_ref[...] = h - m - lse


def _tc_out(acc, ts, dis_col, b3p):
    blk = 1000
    return pl.pallas_call(
        _out_body,
        grid=(NN // blk,),
        in_specs=[
            pl.BlockSpec((1, blk, CP), lambda i: (0, i, 0)),
            pl.BlockSpec((1, blk, CP), lambda i: (1, i, 0)),
            pl.BlockSpec((blk, CP), lambda i: (i, 0)),
            pl.BlockSpec((blk, 1), lambda i: (i, 0)),
            pl.BlockSpec((1, CP), lambda i: (0, 0)),
        ],
        out_specs=pl.BlockSpec((blk, CC), lambda i: (i, 0)),
        out_shape=jax.ShapeDtypeStruct((NN, CC), jnp.float32),
    )(acc, acc, ts, dis_col, b3p)


def kernel(x, edge_index, W1, b1, W2, b2, W3, b3):
    src = edge_index[0].astype(jnp.int32)
    dst = edge_index[1].astype(jnp.int32)

    npad = EP - EE
    pidx = jnp.arange(npad, dtype=jnp.int32)
    pad_src = (pidx * 37) % NN
    pad_dst = NN + (pidx % (NP - NN))
    srcp = jnp.concatenate([src, pad_src])
    dstp = jnp.concatenate([dst, pad_dst])
    dstp2 = dstp.reshape(EP // CHK, CHK)

    zeros_hist = jnp.zeros((NP,), jnp.float32)
    zeros128 = jnp.zeros((NP, HH), jnp.float32)
    zeros64 = jnp.zeros((NP, CP), jnp.float32)

    hist = _degree_hist(dstp, zeros_hist)
    dis_col, ts1 = _tc1(hist, x, W1)

    acc1 = _aggregate(ts1, srcp, dstp2, zeros128, HH)
    ts2 = _tc_mid(acc1, ts1, dis_col, b1.reshape(1, HH), W2, relu=True)
    acc2 = _aggregate(ts2, srcp, dstp2, zeros128, HH)
    W3p = jnp.pad(W3, ((0, 0), (0, CP - CC)))
    b3p = jnp.pad(b3, (0, CP - CC)).reshape(1, CP)
    ts3 = _tc_mid(acc2, ts2, dis_col, b2.reshape(1, HH), W3p, relu=False)
    acc3 = _aggregate(ts3, srcp, dstp2, zeros64, CP)
    return _tc_out(acc3, ts3, dis_col, b3p)


# speedup vs baseline: 1.0063x; 1.0063x over previous
import functools

import jax
import jax.numpy as jnp
from jax import lax
from jax.experimental import pallas as pl
from jax.experimental.pallas import tpu as pltpu
from jax.experimental.pallas import tpu_sc as plsc

NN = 10000
DD = 128
HH = 128
CC = 40
CP = 64

NC = 2
NS = 16
NWK = NC * NS
CHK = 64
NP = 10240
EE = 320000
NCH = 160
EPW = NCH * CHK
EP = EPW * NWK
HIST_R = NP // 16


def _sc_mesh():
    return plsc.VectorSubcoreMesh(core_axis_name="c", subcore_axis_name="s",
                                  num_cores=NC, num_subcores=NS)


def _deg_body(dst_hbm, zero_hbm, out_hbm, dst_v, hist_v, sem):
    cid = lax.axis_index("c")
    sid = lax.axis_index("s")
    wid = cid * NS + sid
    pltpu.sync_copy(dst_hbm.at[pl.ds(wid * EPW, EPW)], dst_v)
    pltpu.sync_copy(zero_hbm, hist_v)
    ones = jnp.ones((16,), jnp.float32)

    def body(i, _):
        idx = dst_v[pl.ds(i * 16, 16)]
        plsc.addupdate_scatter(hist_v, [idx], ones)
        return 0

    lax.fori_loop(0, EPW // 16, body, 0)
    pltpu.sync_copy(hist_v, out_hbm.at[wid])


def _degree_hist(dstp, zeros_hist):
    k = pl.kernel(
        _deg_body,
        out_type=jax.ShapeDtypeStruct((NWK, NP), jnp.float32),
        mesh=_sc_mesh(),
        compiler_params=pltpu.CompilerParams(needs_layout_passes=False),
        scratch_types=[
            pltpu.VMEM((EPW,), jnp.int32),
            pltpu.VMEM((NP,), jnp.float32),
            pltpu.SemaphoreType.DMA,
        ],
    )
    return k(dstp, zeros_hist)


NBUF = 4
NDB = 8


def _agg_body(width, table_hbm, src_hbm, dst2_hbm, zero_hbm, out_hbm,
              acc_sh, src_v, dst_v, rows_v, *sems):
    cid = lax.axis_index("c")
    sid = lax.axis_index("s")
    wid = cid * NS + sid
    rows_per_tile = NP // NS
    gsem = sems[:NBUF]
    ssem = sems[NBUF:2 * NBUF]
    dsem = sems[2 * NBUF:]

    pltpu.sync_copy(src_hbm.at[pl.ds(wid * EPW, EPW)], src_v)

    def start_didx(ch, b):
        pltpu.async_copy(dst2_hbm.at[pl.ds(wid * NCH + ch, 1)],
                         dst_v.at[pl.ds(b, 1)], dsem[b])

    def wait_didx(ch, b):
        pltpu.make_async_copy(dst2_hbm.at[pl.ds(wid * NCH + ch, 1)],
                              dst_v.at[pl.ds(b, 1)], dsem[b]).wait()

    def start_gather(ch, b):
        pltpu.async_copy(table_hbm.at[src_v.at[pl.ds(ch * CHK, CHK)]],
                         rows_v.at[b], gsem[b])

    def wait_gather(ch, b):
        pltpu.make_async_copy(table_hbm.at[src_v.at[pl.ds(ch * CHK, CHK)]],
                              rows_v.at[b], gsem[b]).wait()

    def start_scatter(ch, db, b):
        pltpu.async_copy(rows_v.at[b], acc_sh.at[dst_v.at[db]], ssem[b],
                         add=True)

    def wait_scatter(ch, db, b):
        pltpu.make_async_copy(rows_v.at[b], acc_sh.at[dst_v.at[db]],
                              ssem[b]).wait()

    for ch in range(NBUF):
        start_didx(ch, ch)
    start_gather(0, 0)
    start_gather(1, 1)
    row0 = sid * rows_per_tile

    @pl.when(cid == 0)
    def _():
        pltpu.sync_copy(table_hbm.at[pl.ds(row0, rows_per_tile)],
                        acc_sh.at[pl.ds(row0, rows_per_tile)])

    @pl.when(cid == 1)
    def _():
        pltpu.sync_copy(zero_hbm.at[pl.ds(row0, rows_per_tile)],
                        acc_sh.at[pl.ds(row0, rows_per_tile)])

    plsc.subcore_barrier()

    def outer(k, _):
        for t in range(NDB):
            j = NDB * k + t
            b = t % NBUF
            b2 = (t + 2) % NBUF

            @pl.when(j + 2 < NCH)
            def _():
                @pl.when(j >= 2)
                def _():
                    wait_scatter(j - 2, (t - 2) % NDB, b2)
                start_gather(j + 2, b2)

            @pl.when(j + NBUF < NCH)
            def _():
                start_didx(j + NBUF, (t + NBUF) % NDB)

            wait_gather(j, b)
            wait_didx(j, t)
            start_scatter(j, t, b)
        return 0

    lax.fori_loop(0, NCH // NDB, outer, 0)
    for ch in range(NCH - NBUF, NCH):
        wait_scatter(ch, ch % NDB, ch % NBUF)

    plsc.subcore_barrier()
    pltpu.sync_copy(acc_sh.at[pl.ds(sid * rows_per_tile, rows_per_tile)],
                    out_hbm.at[cid, pl.ds(sid * rows_per_tile, rows_per_tile)])


def _aggregate(table, srcp, dstp2, zeros_acc, width):
    k = pl.kernel(
        functools.partial(_agg_body, width),
        out_type=jax.ShapeDtypeStruct((NC, NP, width), jnp.float32),
        mesh=_sc_mesh(),
        compiler_params=pltpu.CompilerParams(
            needs_layout_passes=False,
            use_tc_tiling_on_sc=(width % 128 == 0)),
        scratch_types=[
            pltpu.VMEM_SHARED((NP, width), jnp.float32),
            pltpu.VMEM((EPW,), jnp.int32),
            pltpu.VMEM((NDB, CHK), jnp.int32),
            pltpu.VMEM((NBUF, CHK, width), jnp.float32),
        ] + [pltpu.SemaphoreType.DMA] * (2 * NBUF + NDB),
    )
    return k(table, srcp, dstp2, zeros_acc)


def _tc1_body(hist_ref, x_ref, w_ref, dis_ref, ts_ref):
    deg = jnp.sum(hist_ref[...], axis=0, keepdims=True) + 1.0
    dis_c = jnp.transpose(lax.rsqrt(deg))
    dis_ref[...] = dis_c
    xw = jnp.dot(x_ref[...], w_ref[...], preferred_element_type=jnp.float32)
    ts_ref[...] = xw * dis_c


def _tc1(hist, x, W1):
    blk = 1024
    return pl.pallas_call(
        _tc1_body,
        grid=(NP // blk,),
        in_specs=[
            pl.BlockSpec((NWK, blk), lambda i: (0, i)),
            pl.BlockSpec((blk, DD), lambda i: (i, 0)),
            pl.BlockSpec((DD, HH), lambda i: (0, 0)),
        ],
        out_specs=[
            pl.BlockSpec((blk, 1), lambda i: (i, 0)),
            pl.BlockSpec((blk, HH), lambda i: (i, 0)),
        ],
        out_shape=[
            jax.ShapeDtypeStruct((NP, 1), jnp.float32),
            jax.ShapeDtypeStruct((NP, HH), jnp.float32),
        ],
    )(hist, x, W1)


def _mid_body(relu, acc0_ref, acc1_ref, dis_ref, b_ref, w_ref, out_ref):
    dis = dis_ref[...]
    h = (acc0_ref[0] + acc1_ref[0]) * dis + b_ref[...]
    if relu:
        h = jnp.maximum(h, 0.0)
    out_ref[...] = jnp.dot(h, w_ref[...],
                           preferred_element_type=jnp.float32) * dis


def _tc_mid(acc, dis_col, b, Wn, relu):
    blk = 1024
    win, wout = Wn.shape
    return pl.pallas_call(
        functools.partial(_mid_body, relu),
        grid=(NP // blk,),
        in_specs=[
            pl.BlockSpec((1, blk, win), lambda i: (0, i, 0)),
            pl.BlockSpec((1, blk, win), lambda i: (1, i, 0)),
            pl.BlockSpec((blk, 1), lambda i: (i, 0)),
            pl.BlockSpec((1, win), lambda i: (0, 0)),
            pl.BlockSpec((win, wout), lambda i: (0, 0)),
        ],
        out_specs=pl.BlockSpec((blk, wout), lambda i: (i, 0)),
        out_shape=jax.ShapeDtypeStruct((NP, wout), jnp.float32),
    )(acc, acc, dis_col, b, Wn)


def _out_body(acc0_ref, acc1_ref, dis_ref, b_ref, out_ref):
    hp = (acc0_ref[0] + acc1_ref[0]) * dis_ref[...] + b_ref[...]
    h = hp[:, :CC]
    m = jnp.max(h, axis=1, keepdims=True)
    e = jnp.exp(h - m)
    lse = jnp.log(jnp.sum(e, axis=1, keepdims=True))
    out_ref[...] = h - m - lse


def _tc_out(acc, dis_col, b3p):
    blk = 1000
    return pl.pallas_call(
        _out_body,
        grid=(NN // blk,),
        in_specs=[
            pl.BlockSpec((1, blk, CP), lambda i: (0, i, 0)),
            pl.BlockSpec((1, blk, CP), lambda i: (1, i, 0)),
            pl.BlockSpec((blk, 1), lambda i: (i, 0)),
            pl.BlockSpec((1, CP), lambda i: (0, 0)),
        ],
        out_specs=pl.BlockSpec((blk, CC), lambda i: (i, 0)),
        out_shape=jax.ShapeDtypeStruct((NN, CC), jnp.float32),
    )(acc, acc, dis_col, b3p)


def kernel(x, edge_index, W1, b1, W2, b2, W3, b3):
    src = edge_index[0].astype(jnp.int32)
    dst = edge_index[1].astype(jnp.int32)

    npad = EP - EE
    pidx = jnp.arange(npad, dtype=jnp.int32)
    pad_src = (pidx * 37) % NN
    pad_dst = NN + (pidx % (NP - NN))
    srcp = jnp.concatenate([src, pad_src])
    dstp = jnp.concatenate([dst, pad_dst])
    dstp2 = dstp.reshape(EP // CHK, CHK)

    zeros_hist = jnp.zeros((NP,), jnp.float32)
    zeros128 = jnp.zeros((NP, HH), jnp.float32)
    zeros64 = jnp.zeros((NP, CP), jnp.float32)

    hist = _degree_hist(dstp, zeros_hist)
    dis_col, ts1 = _tc1(hist, x, W1)

    acc1 = _aggregate(ts1, srcp, dstp2, zeros128, HH)
    ts2 = _tc_mid(acc1, dis_col, b1.reshape(1, HH), W2, relu=True)
    acc2 = _aggregate(ts2, srcp, dstp2, zeros128, HH)
    W3p = jnp.pad(W3, ((0, 0), (0, CP - CC)))
    b3p = jnp.pad(b3, (0, CP - CC)).reshape(1, CP)
    ts3 = _tc_mid(acc2, dis_col, b2.reshape(1, HH), W3p, relu=False)
    acc3 = _aggregate(ts3, srcp, dstp2, zeros64, CP)
    return _tc_out(acc3, dis_col, b3p)


# speedup vs baseline: 1.0065x; 1.0002x over previous
import functools

import jax
import jax.numpy as jnp
from jax import lax
from jax.experimental import pallas as pl
from jax.experimental.pallas import tpu as pltpu
from jax.experimental.pallas import tpu_sc as plsc

NN = 10000
DD = 128
HH = 128
CC = 40
CP = 64

NC = 2
NS = 16
NWK = NC * NS
CHK = 64
NP = 10240
EE = 320000
NCH = 160
EPW = NCH * CHK
EP = EPW * NWK


def _sc_mesh():
    return plsc.VectorSubcoreMesh(core_axis_name="c", subcore_axis_name="s",
                                  num_cores=NC, num_subcores=NS)


def _deg_body(dst_hbm, zero_hbm, out_hbm, dst_v, hist_v, sem):
    cid = lax.axis_index("c")
    sid = lax.axis_index("s")
    wid = cid * NS + sid
    pltpu.sync_copy(dst_hbm.at[pl.ds(wid * EPW, EPW)], dst_v)
    pltpu.sync_copy(zero_hbm, hist_v)
    ones = jnp.ones((16,), jnp.float32)

    def body(i, _):
        idx = dst_v[pl.ds(i * 16, 16)]
        plsc.addupdate_scatter(hist_v, [idx], ones)
        return 0

    lax.fori_loop(0, EPW // 16, body, 0)
    pltpu.sync_copy(hist_v, out_hbm.at[wid])


def _degree_hist(dstp, zeros_hist):
    k = pl.kernel(
        _deg_body,
        out_type=jax.ShapeDtypeStruct((NWK, NP), jnp.float32),
        mesh=_sc_mesh(),
        compiler_params=pltpu.CompilerParams(needs_layout_passes=False),
        scratch_types=[
            pltpu.VMEM((EPW,), jnp.int32),
            pltpu.VMEM((NP,), jnp.float32),
            pltpu.SemaphoreType.DMA,
        ],
    )
    return k(dstp, zeros_hist)


NBUF = 4
NDB = 8


def _agg_body(width, table_hbm, src_hbm, dst2_hbm, zero_hbm, out_hbm,
              acc_sh, src_v, dst_v, rows_v, *sems):
    cid = lax.axis_index("c")
    sid = lax.axis_index("s")
    wid = cid * NS + sid
    rows_per_tile = NP // NS
    gsem = sems[:NBUF]
    ssem = sems[NBUF:2 * NBUF]
    dsem = sems[2 * NBUF:]

    pltpu.sync_copy(src_hbm.at[pl.ds(wid * EPW, EPW)], src_v)

    def start_didx(ch, b):
        pltpu.async_copy(dst2_hbm.at[pl.ds(wid * NCH + ch, 1)],
                         dst_v.at[pl.ds(b, 1)], dsem[b])

    def wait_didx(ch, b):
        pltpu.make_async_copy(dst2_hbm.at[pl.ds(wid * NCH + ch, 1)],
                              dst_v.at[pl.ds(b, 1)], dsem[b]).wait()

    def start_gather(ch, b):
        pltpu.async_copy(table_hbm.at[src_v.at[pl.ds(ch * CHK, CHK)]],
                         rows_v.at[b], gsem[b])

    def wait_gather(ch, b):
        pltpu.make_async_copy(table_hbm.at[src_v.at[pl.ds(ch * CHK, CHK)]],
                              rows_v.at[b], gsem[b]).wait()

    def start_scatter(ch, db, b):
        pltpu.async_copy(rows_v.at[b], acc_sh.at[dst_v.at[db]], ssem[b],
                         add=True)

    def wait_scatter(ch, db, b):
        pltpu.make_async_copy(rows_v.at[b], acc_sh.at[dst_v.at[db]],
                              ssem[b]).wait()

    for ch in range(NBUF):
        start_didx(ch, ch)
    start_gather(0, 0)
    start_gather(1, 1)
    row0 = sid * rows_per_tile

    @pl.when(cid == 0)
    def _():
        pltpu.sync_copy(table_hbm.at[pl.ds(row0, rows_per_tile)],
                        acc_sh.at[pl.ds(row0, rows_per_tile)])

    @pl.when(cid == 1)
    def _():
        pltpu.sync_copy(zero_hbm.at[pl.ds(row0, rows_per_tile)],
                        acc_sh.at[pl.ds(row0, rows_per_tile)])

    plsc.subcore_barrier()

    def outer(k, _):
        for t in range(NDB):
            j = NDB * k + t
            b = t % NBUF
            b2 = (t + 2) % NBUF

            @pl.when(j + 2 < NCH)
            def _():
                @pl.when(j >= 2)
                def _():
                    wait_scatter(j - 2, (t - 2) % NDB, b2)
                start_gather(j + 2, b2)

            @pl.when(j + NBUF < NCH)
            def _():
                start_didx(j + NBUF, (t + NBUF) % NDB)

            wait_gather(j, b)
            wait_didx(j, t)
            start_scatter(j, t, b)
        return 0

    lax.fori_loop(0, NCH // NDB, outer, 0)
    for ch in range(NCH - NBUF, NCH):
        wait_scatter(ch, ch % NDB, ch % NBUF)

    plsc.subcore_barrier()
    pltpu.sync_copy(acc_sh.at[pl.ds(sid * rows_per_tile, rows_per_tile)],
                    out_hbm.at[cid, pl.ds(sid * rows_per_tile, rows_per_tile)])


def _aggregate(table, srcp, dstp2, zeros_acc, width):
    k = pl.kernel(
        functools.partial(_agg_body, width),
        out_type=jax.ShapeDtypeStruct((NC, NP, width), jnp.float32),
        mesh=_sc_mesh(),
        compiler_params=pltpu.CompilerParams(
            needs_layout_passes=False,
            use_tc_tiling_on_sc=(width % 128 == 0)),
        scratch_types=[
            pltpu.VMEM_SHARED((NP, width), jnp.float32),
            pltpu.VMEM((EPW,), jnp.int32),
            pltpu.VMEM((NDB, CHK), jnp.int32),
            pltpu.VMEM((NBUF, CHK, width), jnp.float32),
        ] + [pltpu.SemaphoreType.DMA] * (2 * NBUF + NDB),
    )
    return k(table, srcp, dstp2, zeros_acc)


def _tc1_body(hist_ref, x_ref, w_ref, dis_ref, ts_ref):
    deg = jnp.sum(hist_ref[...], axis=0, keepdims=True) + 1.0
    dis_c = jnp.transpose(lax.rsqrt(deg))
    dis_ref[...] = dis_c
    xw = jnp.dot(x_ref[...], w_ref[...], preferred_element_type=jnp.float32)
    ts_ref[...] = xw * dis_c


def _tc1(hist, x, W1):
    blk = 1024
    return pl.pallas_call(
        _tc1_body,
        grid=(NP // blk,),
        in_specs=[
            pl.BlockSpec((NWK, blk), lambda i: (0, i)),
            pl.BlockSpec((blk, DD), lambda i: (i, 0)),
            pl.BlockSpec((DD, HH), lambda i: (0, 0)),
        ],
        out_specs=[
            pl.BlockSpec((blk, 1), lambda i: (i, 0)),
            pl.BlockSpec((blk, HH), lambda i: (i, 0)),
        ],
        out_shape=[
            jax.ShapeDtypeStruct((NP, 1), jnp.float32),
            jax.ShapeDtypeStruct((NP, HH), jnp.float32),
        ],
    )(hist, x, W1)


def _mid_body(relu, acc0_ref, acc1_ref, dis_ref, b_ref, w_ref, out_ref):
    dis = dis_ref[...]
    h = (acc0_ref[0] + acc1_ref[0]) * dis + b_ref[...]
    if relu:
        h = jnp.maximum(h, 0.0)
    out_ref[...] = jnp.dot(h, w_ref[...],
                           preferred_element_type=jnp.float32) * dis


def _tc_mid(acc, dis_col, b, Wn, relu):
    blk = 1024
    win, wout = Wn.shape
    return pl.pallas_call(
        functools.partial(_mid_body, relu),
        grid=(NP // blk,),
        in_specs=[
            pl.BlockSpec((1, blk, win), lambda i: (0, i, 0)),
            pl.BlockSpec((1, blk, win), lambda i: (1, i, 0)),
            pl.BlockSpec((blk, 1), lambda i: (i, 0)),
            pl.BlockSpec((1, win), lambda i: (0, 0)),
            pl.BlockSpec((win, wout), lambda i: (0, 0)),
        ],
        out_specs=pl.BlockSpec((blk, wout), lambda i: (i, 0)),
        out_shape=jax.ShapeDtypeStruct((NP, wout), jnp.float32),
    )(acc, acc, dis_col, b, Wn)


def _out_body(acc0_ref, acc1_ref, dis_ref, b_ref, out_ref):
    hp = (acc0_ref[0] + acc1_ref[0]) * dis_ref[...] + b_ref[...]
    h = hp[:, :CC]
    m = jnp.max(h, axis=1, keepdims=True)
    e = jnp.exp(h - m)
    lse = jnp.log(jnp.sum(e, axis=1, keepdims=True))
    out_ref[...] = h - m - lse


def _tc_out(acc, dis_col, b3p):
    blk = 1000
    return pl.pallas_call(
        _out_body,
        grid=(NN // blk,),
        in_specs=[
            pl.BlockSpec((1, blk, CP), lambda i: (0, i, 0)),
            pl.BlockSpec((1, blk, CP), lambda i: (1, i, 0)),
            pl.BlockSpec((blk, 1), lambda i: (i, 0)),
            pl.BlockSpec((1, CP), lambda i: (0, 0)),
        ],
        out_specs=pl.BlockSpec((blk, CC), lambda i: (i, 0)),
        out_shape=jax.ShapeDtypeStruct((NN, CC), jnp.float32),
    )(acc, acc, dis_col, b3p)


def kernel(x, edge_index, W1, b1, W2, b2, W3, b3):
    src = edge_index[0].astype(jnp.int32)
    dst = edge_index[1].astype(jnp.int32)

    npad = EP - EE
    pidx = jnp.arange(npad, dtype=jnp.int32)
    pad_src = (pidx * 37) % NN
    pad_dst = NN + (pidx % (NP - NN))
    srcp = jnp.concatenate([src, pad_src])
    dstp = jnp.concatenate([dst, pad_dst])
    dstp2 = dstp.reshape(EP // CHK, CHK)

    zeros_hist = jnp.zeros((NP,), jnp.float32)
    zeros128 = jnp.zeros((NP, HH), jnp.float32)
    zeros64 = jnp.zeros((NP, CP), jnp.float32)

    hist = _degree_hist(dstp, zeros_hist)
    dis_col, ts1 = _tc1(hist, x, W1)

    acc1 = _aggregate(ts1, srcp, dstp2, zeros128, HH)
    ts2 = _tc_mid(acc1, dis_col, b1.reshape(1, HH), W2, relu=True)
    acc2 = _aggregate(ts2, srcp, dstp2, zeros128, HH)
    W3p = jnp.pad(W3, ((0, 0), (0, CP - CC)))
    b3p = jnp.pad(b3, (0, CP - CC)).reshape(1, CP)
    ts3 = _tc_mid(acc2, dis_col, b2.reshape(1, HH), W3p, relu=False)
    acc3 = _aggregate(ts3, srcp, dstp2, zeros64, CP)
    return _tc_out(acc3, dis_col, b3p)


# speedup vs baseline: 1.0163x; 1.0098x over previous
import functools

import jax
import jax.numpy as jnp
from jax import lax
from jax.experimental import pallas as pl
from jax.experimental.pallas import tpu as pltpu
from jax.experimental.pallas import tpu_sc as plsc

NN = 10000
DD = 128
HH = 128
CC = 40
CP = 64

NC = 2
NS = 16
NWK = NC * NS
CHK = 64
NP = 10240
EE = 320000
NCH = 157
EPW = NCH * CHK
EP = EPW * NWK


def _sc_mesh():
    return plsc.VectorSubcoreMesh(core_axis_name="c", subcore_axis_name="s",
                                  num_cores=NC, num_subcores=NS)


def _deg_body(dst_hbm, zero_hbm, out_hbm, dst_v, hist_v, sem):
    cid = lax.axis_index("c")
    sid = lax.axis_index("s")
    wid = cid * NS + sid
    pltpu.sync_copy(dst_hbm.at[pl.ds(wid * EPW, EPW)], dst_v)
    pltpu.sync_copy(zero_hbm, hist_v)
    ones = jnp.ones((16,), jnp.float32)

    def body(i, _):
        idx = dst_v[pl.ds(i * 16, 16)]
        plsc.addupdate_scatter(hist_v, [idx], ones)
        return 0

    lax.fori_loop(0, EPW // 16, body, 0)
    pltpu.sync_copy(hist_v, out_hbm.at[wid])


def _degree_hist(dstp, zeros_hist):
    k = pl.kernel(
        _deg_body,
        out_type=jax.ShapeDtypeStruct((NWK, NP), jnp.float32),
        mesh=_sc_mesh(),
        compiler_params=pltpu.CompilerParams(needs_layout_passes=False),
        scratch_types=[
            pltpu.VMEM((EPW,), jnp.int32),
            pltpu.VMEM((NP,), jnp.float32),
            pltpu.SemaphoreType.DMA,
        ],
    )
    return k(dstp, zeros_hist)


NBUF = 4
NDB = 8


def _agg_body(width, table_hbm, src_hbm, dst2_hbm, zero_hbm, out_hbm,
              acc_sh, src_v, dst_v, rows_v, *sems):
    cid = lax.axis_index("c")
    sid = lax.axis_index("s")
    wid = cid * NS + sid
    rows_per_tile = NP // NS
    gsem = sems[:NBUF]
    ssem = sems[NBUF:2 * NBUF]
    dsem = sems[2 * NBUF:]

    pltpu.sync_copy(src_hbm.at[pl.ds(wid * EPW, EPW)], src_v)

    def start_didx(ch, b):
        pltpu.async_copy(dst2_hbm.at[pl.ds(wid * NCH + ch, 1)],
                         dst_v.at[pl.ds(b, 1)], dsem[b])

    def wait_didx(ch, b):
        pltpu.make_async_copy(dst2_hbm.at[pl.ds(wid * NCH + ch, 1)],
                              dst_v.at[pl.ds(b, 1)], dsem[b]).wait()

    def start_gather(ch, b):
        pltpu.async_copy(table_hbm.at[src_v.at[pl.ds(ch * CHK, CHK)]],
                         rows_v.at[b], gsem[b])

    def wait_gather(ch, b):
        pltpu.make_async_copy(table_hbm.at[src_v.at[pl.ds(ch * CHK, CHK)]],
                              rows_v.at[b], gsem[b]).wait()

    def start_scatter(ch, db, b):
        pltpu.async_copy(rows_v.at[b], acc_sh.at[dst_v.at[db]], ssem[b],
                         add=True)

    def wait_scatter(ch, db, b):
        pltpu.make_async_copy(rows_v.at[b], acc_sh.at[dst_v.at[db]],
                              ssem[b]).wait()

    for ch in range(NBUF):
        start_didx(ch, ch)
    start_gather(0, 0)
    start_gather(1, 1)
    row0 = sid * rows_per_tile

    @pl.when(cid == 0)
    def _():
        pltpu.sync_copy(table_hbm.at[pl.ds(row0, rows_per_tile)],
                        acc_sh.at[pl.ds(row0, rows_per_tile)])

    @pl.when(cid == 1)
    def _():
        pltpu.sync_copy(zero_hbm.at[pl.ds(row0, rows_per_tile)],
                        acc_sh.at[pl.ds(row0, rows_per_tile)])

    plsc.subcore_barrier()

    def step(j, t, guard):
        b = t % NBUF
        b2 = (t + 2) % NBUF
        if guard(j + 2):
            @pl.when(j + 2 < NCH)
            def _():
                @pl.when(j >= 2)
                def _():
                    wait_scatter(j - 2, (t - 2) % NDB, b2)
                start_gather(j + 2, b2)
        elif j >= 2:
            wait_scatter(j - 2, (t - 2) % NDB, b2)

        if guard(j + NBUF):
            @pl.when(j + NBUF < NCH)
            def _():
                start_didx(j + NBUF, (t + NBUF) % NDB)

        wait_gather(j, b)
        wait_didx(j, t)
        start_scatter(j, t, b)

    def outer(k, _):
        for t in range(NDB):
            step(NDB * k + t, t, guard=lambda _: True)
        return 0

    lax.fori_loop(0, NCH // NDB, outer, 0)
    for j in range((NCH // NDB) * NDB, NCH):
        step(j, j % NDB, guard=lambda ch: ch < NCH)
    for ch in range(NCH - 2, NCH):
        wait_scatter(ch, ch % NDB, ch % NBUF)

    plsc.subcore_barrier()
    pltpu.sync_copy(acc_sh.at[pl.ds(sid * rows_per_tile, rows_per_tile)],
                    out_hbm.at[cid, pl.ds(sid * rows_per_tile, rows_per_tile)])


def _aggregate(table, srcp, dstp2, zeros_acc, width):
    k = pl.kernel(
        functools.partial(_agg_body, width),
        out_type=jax.ShapeDtypeStruct((NC, NP, width), jnp.float32),
        mesh=_sc_mesh(),
        compiler_params=pltpu.CompilerParams(
            needs_layout_passes=False,
            use_tc_tiling_on_sc=(width % 128 == 0)),
        scratch_types=[
            pltpu.VMEM_SHARED((NP, width), jnp.float32),
            pltpu.VMEM((EPW,), jnp.int32),
            pltpu.VMEM((NDB, CHK), jnp.int32),
            pltpu.VMEM((NBUF, CHK, width), jnp.float32),
        ] + [pltpu.SemaphoreType.DMA] * (2 * NBUF + NDB),
    )
    return k(table, srcp, dstp2, zeros_acc)


def _tc1_body(hist_ref, x_ref, w_ref, dis_ref, ts_ref):
    deg = jnp.sum(hist_ref[...], axis=0, keepdims=True) + 1.0
    dis_c = jnp.transpose(lax.rsqrt(deg))
    dis_ref[...] = dis_c
    xw = jnp.dot(x_ref[...], w_ref[...], preferred_element_type=jnp.float32)
    ts_ref[...] = xw * dis_c


def _tc1(hist, x, W1):
    blk = 1024
    return pl.pallas_call(
        _tc1_body,
        grid=(NP // blk,),
        in_specs=[
            pl.BlockSpec((NWK, blk), lambda i: (0, i)),
            pl.BlockSpec((blk, DD), lambda i: (i, 0)),
            pl.BlockSpec((DD, HH), lambda i: (0, 0)),
        ],
        out_specs=[
            pl.BlockSpec((blk, 1), lambda i: (i, 0)),
            pl.BlockSpec((blk, HH), lambda i: (i, 0)),
        ],
        out_shape=[
            jax.ShapeDtypeStruct((NP, 1), jnp.float32),
            jax.ShapeDtypeStruct((NP, HH), jnp.float32),
        ],
    )(hist, x, W1)


def _mid_body(relu, acc0_ref, acc1_ref, dis_ref, b_ref, w_ref, out_ref):
    dis = dis_ref[...]
    h = (acc0_ref[0] + acc1_ref[0]) * dis + b_ref[...]
    if relu:
        h = jnp.maximum(h, 0.0)
    out_ref[...] = jnp.dot(h, w_ref[...],
                           preferred_element_type=jnp.float32) * dis


def _tc_mid(acc, dis_col, b, Wn, relu):
    blk = 1024
    win, wout = Wn.shape
    return pl.pallas_call(
        functools.partial(_mid_body, relu),
        grid=(NP // blk,),
        in_specs=[
            pl.BlockSpec((1, blk, win), lambda i: (0, i, 0)),
            pl.BlockSpec((1, blk, win), lambda i: (1, i, 0)),
            pl.BlockSpec((blk, 1), lambda i: (i, 0)),
            pl.BlockSpec((1, win), lambda i: (0, 0)),
            pl.BlockSpec((win, wout), lambda i: (0, 0)),
        ],
        out_specs=pl.BlockSpec((blk, wout), lambda i: (i, 0)),
        out_shape=jax.ShapeDtypeStruct((NP, wout), jnp.float32),
    )(acc, acc, dis_col, b, Wn)


def _out_body(acc0_ref, acc1_ref, dis_ref, b_ref, out_ref):
    hp = (acc0_ref[0] + acc1_ref[0]) * dis_ref[...] + b_ref[...]
    h = hp[:, :CC]
    m = jnp.max(h, axis=1, keepdims=True)
    e = jnp.exp(h - m)
    lse = jnp.log(jnp.sum(e, axis=1, keepdims=True))
    out_ref[...] = h - m - lse


def _tc_out(acc, dis_col, b3p):
    blk = 1000
    return pl.pallas_call(
        _out_body,
        grid=(NN // blk,),
        in_specs=[
            pl.BlockSpec((1, blk, CP), lambda i: (0, i, 0)),
            pl.BlockSpec((1, blk, CP), lambda i: (1, i, 0)),
            pl.BlockSpec((blk, 1), lambda i: (i, 0)),
            pl.BlockSpec((1, CP), lambda i: (0, 0)),
        ],
        out_specs=pl.BlockSpec((blk, CC), lambda i: (i, 0)),
        out_shape=jax.ShapeDtypeStruct((NN, CC), jnp.float32),
    )(acc, acc, dis_col, b3p)


def kernel(x, edge_index, W1, b1, W2, b2, W3, b3):
    src = edge_index[0].astype(jnp.int32)
    dst = edge_index[1].astype(jnp.int32)

    npad = EP - EE
    pidx = jnp.arange(npad, dtype=jnp.int32)
    pad_src = (pidx * 37) % NN
    pad_dst = NN + (pidx % (NP - NN))
    srcp = jnp.concatenate([src, pad_src])
    dstp = jnp.concatenate([dst, pad_dst])
    dstp2 = dstp.reshape(EP // CHK, CHK)

    zeros_hist = jnp.zeros((NP,), jnp.float32)
    zeros128 = jnp.zeros((NP, HH), jnp.float32)
    zeros64 = jnp.zeros((NP, CP), jnp.float32)

    hist = _degree_hist(dstp, zeros_hist)
    dis_col, ts1 = _tc1(hist, x, W1)

    acc1 = _aggregate(ts1, srcp, dstp2, zeros128, HH)
    ts2 = _tc_mid(acc1, dis_col, b1.reshape(1, HH), W2, relu=True)
    acc2 = _aggregate(ts2, srcp, dstp2, zeros128, HH)
    W3p = jnp.pad(W3, ((0, 0), (0, CP - CC)))
    b3p = jnp.pad(b3, (0, CP - CC)).reshape(1, CP)
    ts3 = _tc_mid(acc2, dis_col, b2.reshape(1, HH), W3p, relu=False)
    acc3 = _aggregate(ts3, srcp, dstp2, zeros64, CP)
    return _tc_out(acc3, dis_col, b3p)
